# Initial kernel scaffold; baseline (speedup 1.0000x reference)
import jax
import jax.numpy as jnp
from jax.experimental import pallas as pl


def kernel(h, cluster_ids, W_down, b_down, W_up, b_up):
    raise NotImplementedError("write your pallas kernel here")


# speedup vs baseline: 2.6120x; 2.6120x over previous
import functools

import jax
import jax.numpy as jnp
from jax import lax
from jax.experimental import pallas as pl

_INV_SQRT2 = 0.7071067811865476


def _adapter_body(cid_ref, h_ref, wd_ref, bd_ref, wu_ref, bu_ref, out_ref, *, bd_dim):
    hb = h_ref[...]
    cid = cid_ref[...]
    h16 = hb.astype(jnp.bfloat16)
    z = jnp.dot(h16, wd_ref[...], preferred_element_type=jnp.float32)
    z = z + bd_ref[...]
    a = 0.5 * z * (1.0 + lax.erf(z * _INV_SQRT2))
    col_expert = lax.broadcasted_iota(jnp.int32, z.shape, 1) // bd_dim
    am = jnp.where(col_expert == cid, a, 0.0).astype(jnp.bfloat16)
    delta = jnp.dot(am, wu_ref[...], preferred_element_type=jnp.float32)
    n_e = bu_ref.shape[0]
    oh = (lax.broadcasted_iota(jnp.int32, (hb.shape[0], n_e), 1) == cid)
    bu_sel = jnp.dot(oh.astype(jnp.float32), bu_ref[...],
                     preferred_element_type=jnp.float32)
    out_ref[...] = hb + delta + bu_sel


def kernel(h, cluster_ids, W_down, b_down, W_up, b_up):
    B, D = h.shape
    E, _, BD = W_down.shape
    BT = 256

    cid2 = cluster_ids.astype(jnp.int32).reshape(B, 1)
    wd_cat = jnp.transpose(W_down, (1, 0, 2)).reshape(D, E * BD).astype(jnp.bfloat16)
    bd_cat = b_down.reshape(1, E * BD)
    wu_cat = W_up.reshape(E * BD, D).astype(jnp.bfloat16)

    return pl.pallas_call(
        functools.partial(_adapter_body, bd_dim=BD),
        grid=(B // BT,),
        in_specs=[
            pl.BlockSpec((BT, 1), lambda i: (i, 0)),
            pl.BlockSpec((BT, D), lambda i: (i, 0)),
            pl.BlockSpec((D, E * BD), lambda i: (0, 0)),
            pl.BlockSpec((1, E * BD), lambda i: (0, 0)),
            pl.BlockSpec((E * BD, D), lambda i: (0, 0)),
            pl.BlockSpec((E, D), lambda i: (0, 0)),
        ],
        out_specs=pl.BlockSpec((BT, D), lambda i: (i, 0)),
        out_shape=jax.ShapeDtypeStruct((B, D), jnp.float32),
    )(cid2, h, wd_cat, bd_cat, wu_cat, b_up)


# speedup vs baseline: 2.8952x; 1.1084x over previous
import functools

import jax
import jax.numpy as jnp
from jax import lax
from jax.experimental import pallas as pl

_INV_SQRT2 = 0.7071067811865476


def _adapter_body(cid_ref, h_ref, wd_ref, bd_ref, wu_ref, bu_ref, out_ref, *, bd_dim):
    hb = h_ref[...]
    cid = cid_ref[...]
    h16 = hb.astype(jnp.bfloat16)
    z = jnp.dot(h16, wd_ref[...], preferred_element_type=jnp.float32)
    z = z + bd_ref[...]
    a = 0.5 * z * (1.0 + lax.erf(z * _INV_SQRT2))
    col_expert = lax.broadcasted_iota(jnp.int32, z.shape, 1) // bd_dim
    am = jnp.where(col_expert == cid, a, 0.0).astype(jnp.bfloat16)
    delta = jnp.dot(am, wu_ref[...], preferred_element_type=jnp.float32)
    n_e = bu_ref.shape[0]
    oh = (lax.broadcasted_iota(jnp.int32, (hb.shape[0], n_e), 1) == cid)
    bu_sel = jnp.dot(oh.astype(jnp.float32), bu_ref[...],
                     preferred_element_type=jnp.float32)
    out_ref[...] = hb + delta + bu_sel


def kernel(h, cluster_ids, W_down, b_down, W_up, b_up):
    B, D = h.shape
    E, _, BD = W_down.shape
    BT = 512

    cid2 = cluster_ids.astype(jnp.int32).reshape(B, 1)
    wd_cat = jnp.transpose(W_down, (1, 0, 2)).reshape(D, E * BD).astype(jnp.bfloat16)
    bd_cat = b_down.reshape(1, E * BD)
    wu_cat = W_up.reshape(E * BD, D).astype(jnp.bfloat16)

    return pl.pallas_call(
        functools.partial(_adapter_body, bd_dim=BD),
        grid=(B // BT,),
        in_specs=[
            pl.BlockSpec((BT, 1), lambda i: (i, 0)),
            pl.BlockSpec((BT, D), lambda i: (i, 0)),
            pl.BlockSpec((D, E * BD), lambda i: (0, 0)),
            pl.BlockSpec((1, E * BD), lambda i: (0, 0)),
            pl.BlockSpec((E * BD, D), lambda i: (0, 0)),
            pl.BlockSpec((E, D), lambda i: (0, 0)),
        ],
        out_specs=pl.BlockSpec((BT, D), lambda i: (i, 0)),
        out_shape=jax.ShapeDtypeStruct((B, D), jnp.float32),
    )(cid2, h, wd_cat, bd_cat, wu_cat, b_up)
